# baseline (speedup 1.0000x reference)
import jax
import jax.numpy as jnp
from jax.experimental import pallas as pl
from jax.experimental.pallas import tpu as pltpu

B, T = 4, 4
S, H = 4096, 768
P = 128
NUM_TRAITS = 12
HH = H // 2
S_BLK = 512


def _gate_modulate_kernel(idx_ref, hs_ref, table_ref, wp_ref, bp_ref,
                          w1_ref, b1_ref, w2_ref, b2_ref, out_ref, gates_ref):
    b = pl.program_id(0)
    s = pl.program_id(1)

    @pl.when((b == 0) & (s == 0))
    def _():
        iota_k = jax.lax.broadcasted_iota(jnp.int32, (B, NUM_TRAITS), 1)
        acc = jnp.zeros((B, NUM_TRAITS), jnp.float32)
        for t in range(T):
            acc = acc + (idx_ref[:, t][:, None] == iota_k).astype(jnp.float32)
        pooled = acc * (1.0 / T)
        pv = jnp.dot(pooled, table_ref[...],
                     preferred_element_type=jnp.float32)
        h = jnp.dot(pv, wp_ref[...],
                    preferred_element_type=jnp.float32) + bp_ref[...]
        g = jnp.tanh(jnp.dot(h, w1_ref[...],
                             preferred_element_type=jnp.float32) + b1_ref[...])
        gates_ref[...] = jax.nn.sigmoid(
            jnp.dot(g, w2_ref[...],
                    preferred_element_type=jnp.float32) + b2_ref[...])

    gate_row = gates_ref[pl.ds(b, 1), :]
    out_ref[...] = hs_ref[...] * gate_row[None]


def kernel(trait_indices, hidden_states, trait_table, W_proj, b_proj,
           W1, b1, W2, b2):
    const = lambda *_: (0, 0)
    grid = (B, S // S_BLK)
    return pl.pallas_call(
        _gate_modulate_kernel,
        grid=grid,
        in_specs=[
            pl.BlockSpec((B, T), const),
            pl.BlockSpec((1, S_BLK, H), lambda b, s: (b, s, 0)),
            pl.BlockSpec((NUM_TRAITS, P), const),
            pl.BlockSpec((P, H), const),
            pl.BlockSpec((1, H), const),
            pl.BlockSpec((H, HH), const),
            pl.BlockSpec((1, HH), const),
            pl.BlockSpec((HH, H), const),
            pl.BlockSpec((1, H), const),
        ],
        out_specs=pl.BlockSpec((1, S_BLK, H), lambda b, s: (b, s, 0)),
        out_shape=jax.ShapeDtypeStruct((B, S, H), jnp.float32),
        scratch_shapes=[pltpu.VMEM((B, H), jnp.float32)],
    )(
        trait_indices.astype(jnp.int32),
        hidden_states,
        trait_table,
        W_proj,
        b_proj.reshape(1, H),
        W1,
        b1.reshape(1, HH),
        W2,
        b2.reshape(1, H),
    )


# speedup vs baseline: 1.1460x; 1.1460x over previous
import jax
import jax.numpy as jnp
from jax.experimental import pallas as pl
from jax.experimental.pallas import tpu as pltpu

B, T = 4, 4
S, H = 4096, 768
P = 128
NUM_TRAITS = 12
HH = H // 2
ROWS = 2048
SPB = S // ROWS


def _gate_kernel(idx_ref, table_ref, wp_ref, bp_ref, w1_ref, b1_ref,
                 w2_ref, b2_ref, gates_ref):
    iota_k = jax.lax.broadcasted_iota(jnp.int32, (B, NUM_TRAITS), 1)
    acc = jnp.zeros((B, NUM_TRAITS), jnp.float32)
    for t in range(T):
        acc = acc + (idx_ref[:, t][:, None] == iota_k).astype(jnp.float32)
    pooled = acc * (1.0 / T)
    pv = jnp.dot(pooled, table_ref[...],
                 preferred_element_type=jnp.float32)
    h = jnp.dot(pv, wp_ref[...],
                preferred_element_type=jnp.float32) + bp_ref[...]
    g = jnp.tanh(jnp.dot(h, w1_ref[...],
                         preferred_element_type=jnp.float32) + b1_ref[...])
    gates_ref[...] = jax.nn.sigmoid(
        jnp.dot(g, w2_ref[...],
                preferred_element_type=jnp.float32) + b2_ref[...])


def _modulate_kernel(gates_ref, hs_ref, out_ref):
    b = pl.program_id(0) // SPB
    gate_row = gates_ref[pl.ds(b, 1), :]
    out_ref[...] = hs_ref[...] * gate_row


def kernel(trait_indices, hidden_states, trait_table, W_proj, b_proj,
           W1, b1, W2, b2):
    whole = lambda *_: (0, 0)
    gates = pl.pallas_call(
        _gate_kernel,
        in_specs=[
            pl.BlockSpec((B, T), whole),
            pl.BlockSpec((NUM_TRAITS, P), whole),
            pl.BlockSpec((P, H), whole),
            pl.BlockSpec((1, H), whole),
            pl.BlockSpec((H, HH), whole),
            pl.BlockSpec((1, HH), whole),
            pl.BlockSpec((HH, H), whole),
            pl.BlockSpec((1, H), whole),
        ],
        out_specs=pl.BlockSpec((B, H), whole),
        out_shape=jax.ShapeDtypeStruct((B, H), jnp.float32),
    )(
        trait_indices.astype(jnp.int32),
        trait_table,
        W_proj,
        b_proj.reshape(1, H),
        W1,
        b1.reshape(1, HH),
        W2,
        b2.reshape(1, H),
    )

    hs2d = hidden_states.reshape(B * S, H)
    out2d = pl.pallas_call(
        _modulate_kernel,
        grid=(B * S // ROWS,),
        in_specs=[
            pl.BlockSpec((B, H), whole),
            pl.BlockSpec((ROWS, H), lambda i: (i, 0)),
        ],
        out_specs=pl.BlockSpec((ROWS, H), lambda i: (i, 0)),
        out_shape=jax.ShapeDtypeStruct((B * S, H), jnp.float32),
    )(gates, hs2d)
    return out2d.reshape(B, S, H)


# speedup vs baseline: 1.1802x; 1.0299x over previous
import jax
import jax.numpy as jnp
from jax.experimental import pallas as pl
from jax.experimental.pallas import tpu as pltpu

B, T = 4, 4
S, H = 4096, 768
P = 128
NUM_TRAITS = 12
HH = H // 2
ROWS = 4096
SPB = S // ROWS


def _gate_kernel(idx_ref, table_ref, wp_ref, bp_ref, w1_ref, b1_ref,
                 w2_ref, b2_ref, gates_ref):
    iota_k = jax.lax.broadcasted_iota(jnp.int32, (B, NUM_TRAITS), 1)
    acc = jnp.zeros((B, NUM_TRAITS), jnp.float32)
    for t in range(T):
        acc = acc + (idx_ref[:, t][:, None] == iota_k).astype(jnp.float32)
    pooled = acc * (1.0 / T)
    pv = jnp.dot(pooled, table_ref[...],
                 preferred_element_type=jnp.float32)
    h = jnp.dot(pv, wp_ref[...],
                preferred_element_type=jnp.float32) + bp_ref[...]
    g = jnp.tanh(jnp.dot(h, w1_ref[...],
                         preferred_element_type=jnp.float32) + b1_ref[...])
    gates_ref[...] = jax.nn.sigmoid(
        jnp.dot(g, w2_ref[...],
                preferred_element_type=jnp.float32) + b2_ref[...])


def _modulate_kernel(gates_ref, hs_ref, out_ref):
    b = pl.program_id(0) // SPB
    gate_row = gates_ref[pl.ds(b, 1), :]
    out_ref[...] = hs_ref[...] * gate_row


def kernel(trait_indices, hidden_states, trait_table, W_proj, b_proj,
           W1, b1, W2, b2):
    whole = lambda *_: (0, 0)
    gates = pl.pallas_call(
        _gate_kernel,
        in_specs=[
            pl.BlockSpec((B, T), whole),
            pl.BlockSpec((NUM_TRAITS, P), whole),
            pl.BlockSpec((P, H), whole),
            pl.BlockSpec((1, H), whole),
            pl.BlockSpec((H, HH), whole),
            pl.BlockSpec((1, HH), whole),
            pl.BlockSpec((HH, H), whole),
            pl.BlockSpec((1, H), whole),
        ],
        out_specs=pl.BlockSpec((B, H), whole),
        out_shape=jax.ShapeDtypeStruct((B, H), jnp.float32),
    )(
        trait_indices.astype(jnp.int32),
        trait_table,
        W_proj,
        b_proj.reshape(1, H),
        W1,
        b1.reshape(1, HH),
        W2,
        b2.reshape(1, H),
    )

    hs2d = hidden_states.reshape(B * S, H)
    out2d = pl.pallas_call(
        _modulate_kernel,
        grid=(B * S // ROWS,),
        in_specs=[
            pl.BlockSpec((B, H), whole),
            pl.BlockSpec((ROWS, H), lambda i: (i, 0)),
        ],
        out_specs=pl.BlockSpec((ROWS, H), lambda i: (i, 0)),
        out_shape=jax.ShapeDtypeStruct((B * S, H), jnp.float32),
    )(gates, hs2d)
    return out2d.reshape(B, S, H)


# speedup vs baseline: 1.2800x; 1.0846x over previous
import jax
import jax.numpy as jnp
from jax.experimental import pallas as pl
from jax.experimental.pallas import tpu as pltpu

B, T = 4, 4
S, H = 4096, 768
P = 128
NUM_TRAITS = 12
HH = H // 2


def _fused_kernel(idx_ref, hs_ref, table_ref, wp_ref, bp_ref,
                  w1_ref, b1_ref, w2_ref, b2_ref, out_ref, gates_ref):
    b = pl.program_id(0)

    @pl.when(b == 0)
    def _():
        iota_k = jax.lax.broadcasted_iota(jnp.int32, (B, NUM_TRAITS), 1)
        acc = jnp.zeros((B, NUM_TRAITS), jnp.float32)
        for t in range(T):
            acc = acc + (idx_ref[:, t][:, None] == iota_k).astype(jnp.float32)
        pooled = acc * (1.0 / T)
        pv = jnp.dot(pooled, table_ref[...],
                     preferred_element_type=jnp.float32)
        h = jnp.dot(pv, wp_ref[...],
                    preferred_element_type=jnp.float32) + bp_ref[...]
        g = jnp.tanh(jnp.dot(h, w1_ref[...],
                             preferred_element_type=jnp.float32) + b1_ref[...])
        gates_ref[...] = jax.nn.sigmoid(
            jnp.dot(g, w2_ref[...],
                    preferred_element_type=jnp.float32) + b2_ref[...])

    gate_row = gates_ref[pl.ds(b, 1), :]
    out_ref[...] = hs_ref[...] * gate_row


def kernel(trait_indices, hidden_states, trait_table, W_proj, b_proj,
           W1, b1, W2, b2):
    whole = lambda *_: (0, 0)
    hs2d = hidden_states.reshape(B * S, H)
    out2d = pl.pallas_call(
        _fused_kernel,
        grid=(B,),
        in_specs=[
            pl.BlockSpec((B, T), whole),
            pl.BlockSpec((S, H), lambda i: (i, 0)),
            pl.BlockSpec((NUM_TRAITS, P), whole),
            pl.BlockSpec((P, H), whole),
            pl.BlockSpec((1, H), whole),
            pl.BlockSpec((H, HH), whole),
            pl.BlockSpec((1, HH), whole),
            pl.BlockSpec((HH, H), whole),
            pl.BlockSpec((1, H), whole),
        ],
        out_specs=pl.BlockSpec((S, H), lambda i: (i, 0)),
        out_shape=jax.ShapeDtypeStruct((B * S, H), jnp.float32),
        scratch_shapes=[pltpu.VMEM((B, H), jnp.float32)],
    )(
        trait_indices.astype(jnp.int32),
        hs2d,
        trait_table,
        W_proj,
        b_proj.reshape(1, H),
        W1,
        b1.reshape(1, HH),
        W2,
        b2.reshape(1, H),
    )
    return out2d.reshape(B, S, H)
